# baseline (speedup 1.0000x reference)
import numpy as np
import jax
import jax.numpy as jnp
from jax import lax
from jax.experimental import pallas as pl
from jax.experimental.pallas import tpu as pltpu

N_BATCH = 16
N_CHAN = 512
N_COL = 4096
NG = N_CHAN // 8


def _body(in_ref, out_ref):
    i = lax.broadcasted_iota(jnp.int32, (8, N_COL), 0)
    bit2 = (i & 2) != 0
    bit1 = (i & 1) != 0
    for j in range(NG):
        g = in_ref[0, (NG - 1 - j) * 8:(NG - j) * 8, :]
        a = pltpu.roll(g, 4, 0)
        b = jnp.where(bit2, pltpu.roll(a, 2, 0), pltpu.roll(a, 6, 0))
        c = jnp.where(bit1, pltpu.roll(b, 1, 0), pltpu.roll(b, 7, 0))
        out_ref[0, j * 8:(j + 1) * 8, :] = c


def kernel(x, cond):
    del cond
    z = pl.pallas_call(
        _body,
        grid=(N_BATCH,),
        in_specs=[
            pl.BlockSpec((1, N_CHAN, N_COL), lambda b: (b, 0, 0)),
        ],
        out_specs=pl.BlockSpec((1, N_CHAN, N_COL), lambda b: (b, 0, 0)),
        out_shape=jax.ShapeDtypeStruct((N_BATCH, N_CHAN, N_COL),
                                       jnp.float32),
    )(x)
    log_det_J = jnp.zeros((1,), dtype=jnp.float32)
    return (z, log_det_J)
